# baseline (speedup 1.0000x reference)
import jax
import jax.numpy as jnp
from jax.experimental import pallas as pl

N = 33
E = 545
T = 512
TB = 16
KB = 2816


def _lrelu(v):
    return jnp.maximum(v, 0.2 * v)


def _elu(v):
    return jnp.where(v > 0, v, jnp.exp(v) - 1.0)


def _gat_body(x_ref, gs_ref, gd_ref, gdt_ref, ea_ref, wl1_ref, wr1_ref,
              we1_ref, a1_ref, r1_ref, b1_ref, wlr2_ref, we2_ref, att2_ref,
              b2_ref, out_ref):
    gs = gs_ref[...]
    gd = gd_ref[...]
    gdt = gdt_ref[...]
    ea = ea_ref[...]
    a1 = a1_ref[...]
    r1 = r1_ref[...]
    att2c = att2_ref[...]
    wl1 = wl1_ref[...]
    wr1 = wr1_ref[...]
    wlr2 = wlr2_ref[...]
    b1 = b1_ref[...]
    b2 = b2_ref[...]

    e1 = (ea[:, 0:1] * we1_ref[0:1, :] + ea[:, 1:2] * we1_ref[1:2, :]
          + ea[:, 2:3] * we1_ref[2:3, :])
    e2 = (ea[:, 0:1] * we2_ref[0:1, :] + ea[:, 1:2] * we2_ref[1:2, :]
          + ea[:, 2:3] * we2_ref[2:3, :])

    def one_t(xt):
        xs = jnp.dot(gs, xt, preferred_element_type=jnp.float32)
        xd = jnp.dot(gd, xt, preferred_element_type=jnp.float32)
        xls = xs[:, 0:1] * wl1[0:1, :] + xs[:, 1:2] * wl1[1:2, :]
        xrd = xd[:, 0:1] * wr1[0:1, :] + xd[:, 1:2] * wr1[1:2, :]
        m1 = _lrelu(xls + xrd + e1)
        al1 = jnp.dot(m1, a1, preferred_element_type=jnp.float32)
        al1t = al1.T
        masked = jnp.where(gdt[:, None, :] > 0.5, al1t[None, :, :], -1e30)
        amax = jnp.max(masked, axis=2)
        amax_d = jnp.dot(gd, amax, preferred_element_type=jnp.float32)
        p1 = jnp.exp(al1 - amax_d)
        s1 = jnp.dot(gdt, p1, preferred_element_type=jnp.float32)
        den1 = jnp.dot(gd, s1, preferred_element_type=jnp.float32)
        aln1 = p1 / (den1 + 1e-16)
        ab1 = jnp.dot(aln1, r1, preferred_element_type=jnp.float32)
        h1 = _elu(jnp.dot(gdt, xls * ab1,
                          preferred_element_type=jnp.float32) + b1)

        lr = jnp.dot(h1, wlr2, preferred_element_type=jnp.float32)
        xls2 = jnp.dot(gs, lr[:, 0:512], preferred_element_type=jnp.float32)
        xrd2 = jnp.dot(gd, lr[:, 512:1024], preferred_element_type=jnp.float32)
        m2 = _lrelu(xls2 + xrd2 + e2)
        al2 = jnp.dot(m2, att2c, preferred_element_type=jnp.float32)
        masked2 = jnp.where(gd > 0.5, al2, -1e30)
        amax2 = jnp.max(masked2, axis=0, keepdims=True).T
        amax2d = jnp.dot(gd, amax2, preferred_element_type=jnp.float32)
        p2 = jnp.exp(al2 - amax2d)
        s2 = jnp.dot(gdt, p2, preferred_element_type=jnp.float32)
        den2 = jnp.dot(gd, s2, preferred_element_type=jnp.float32)
        aln2 = p2 / (den2 + 1e-16)
        h2 = _elu(jnp.dot(gdt, xls2 * aln2,
                          preferred_element_type=jnp.float32) + b2)
        return h2

    def per_pair(i, carry):
        t = 2 * i
        out_ref[t] = one_t(x_ref[t])
        out_ref[t + 1] = one_t(x_ref[t + 1])
        return carry

    jax.lax.fori_loop(0, TB // 2, per_pair, 0)


def _proj_body(xl_ref, w_ref, b_ref, out_ref):
    k = pl.program_id(0)
    part = jnp.dot(xl_ref[...], w_ref[...], preferred_element_type=jnp.float32)

    @pl.when(k == 0)
    def _():
        out_ref[...] = part + b_ref[...]

    @pl.when(k > 0)
    def _():
        out_ref[...] += part


def _lstm_body(xp_ref, whh1_ref, wih2_ref, whh2_ref, b2_ref, wfc_ref,
               bfc_ref, out_ref):
    whh1 = whh1_ref[...]
    wih2 = wih2_ref[...]
    whh2 = whh2_ref[...]
    b2 = b2_ref[...]

    def outer(t8, carry):
        h1, c1, h2, c2 = carry
        blk = xp_ref[t8]
        for j in range(8):
            xt = blk[j:j + 1, :]
            g1 = xt + jnp.dot(h1, whh1, preferred_element_type=jnp.float32)
            i1 = jax.nn.sigmoid(g1[:, 0:128])
            f1 = jax.nn.sigmoid(g1[:, 128:256])
            gg1 = jnp.tanh(g1[:, 256:384])
            o1 = jax.nn.sigmoid(g1[:, 384:512])
            c1 = f1 * c1 + i1 * gg1
            h1 = o1 * jnp.tanh(c1)
            g2 = (jnp.dot(h1, wih2, preferred_element_type=jnp.float32)
                  + jnp.dot(h2, whh2, preferred_element_type=jnp.float32) + b2)
            i2 = jax.nn.sigmoid(g2[:, 0:64])
            f2 = jax.nn.sigmoid(g2[:, 64:128])
            gg2 = jnp.tanh(g2[:, 128:192])
            o2 = jax.nn.sigmoid(g2[:, 192:256])
            c2 = f2 * c2 + i2 * gg2
            h2 = o2 * jnp.tanh(c2)
        return (h1, c1, h2, c2)

    z1 = jnp.zeros((1, 128), jnp.float32)
    z2 = jnp.zeros((1, 64), jnp.float32)
    _, _, h2, _ = jax.lax.fori_loop(0, T // 8, outer, (z1, z1, z2, z2))
    out_ref[...] = (jnp.dot(h2, wfc_ref[...],
                            preferred_element_type=jnp.float32) + bfc_ref[...])


def _run(x, Gs, Gd, ea, Wl1, Wr1, We1, A1, R1, b1, Wlr2, We2, att2, b2,
         Wih1, bsum1, Whh1, Wih2, Whh2, bsum2, Wfc, bfc, interpret=False):
    const = lambda *_: tuple(0 for _ in range(2))
    X = pl.pallas_call(
        _gat_body,
        grid=(T // TB,),
        in_specs=[
            pl.BlockSpec((TB, N, 2), lambda i: (i, 0, 0)),
            pl.BlockSpec((E, N), lambda i: (0, 0)),
            pl.BlockSpec((E, N), lambda i: (0, 0)),
            pl.BlockSpec((N, E), lambda i: (0, 0)),
            pl.BlockSpec((E, 3), lambda i: (0, 0)),
            pl.BlockSpec((2, 512), lambda i: (0, 0)),
            pl.BlockSpec((2, 512), lambda i: (0, 0)),
            pl.BlockSpec((3, 512), lambda i: (0, 0)),
            pl.BlockSpec((512, 8), lambda i: (0, 0)),
            pl.BlockSpec((8, 512), lambda i: (0, 0)),
            pl.BlockSpec((1, 512), lambda i: (0, 0)),
            pl.BlockSpec((512, 1024), lambda i: (0, 0)),
            pl.BlockSpec((3, 512), lambda i: (0, 0)),
            pl.BlockSpec((512, 1), lambda i: (0, 0)),
            pl.BlockSpec((1, 512), lambda i: (0, 0)),
        ],
        out_specs=pl.BlockSpec((TB, N, 512), lambda i: (i, 0, 0)),
        out_shape=jax.ShapeDtypeStruct((T, N, 512), jnp.float32),
        interpret=interpret,
    )(x, Gs, Gd, Gd.T, ea, Wl1, Wr1, We1, A1, R1, b1, Wlr2, We2,
      att2.reshape(512, 1), b2)

    Xl = jnp.transpose(X, (1, 0, 2)).reshape(T, N * 512)

    Xp = pl.pallas_call(
        _proj_body,
        grid=(N * 512 // KB,),
        in_specs=[
            pl.BlockSpec((T, KB), lambda k: (0, k)),
            pl.BlockSpec((KB, 512), lambda k: (k, 0)),
            pl.BlockSpec((1, 512), lambda k: (0, 0)),
        ],
        out_specs=pl.BlockSpec((T, 512), lambda k: (0, 0)),
        out_shape=jax.ShapeDtypeStruct((T, 512), jnp.float32),
        interpret=interpret,
    )(Xl, Wih1, bsum1)

    out = pl.pallas_call(
        _lstm_body,
        interpret=interpret,
        out_shape=jax.ShapeDtypeStruct((1, 10), jnp.float32),
    )(Xp.reshape(T // 8, 8, 512), Whh1, Wih2, Whh2, bsum2, Wfc, bfc)
    return out


def kernel(x, edge_index, edge_attr, Wl1, Wr1, We1, att1, b1, Wl2, Wr2, We2,
           att2, b2, Wih1, Whh1, bih1, bhh1, Wih2, Whh2, bih2, bhh2, Wfc,
           bfc):
    loop = jnp.arange(N, dtype=edge_index.dtype)
    src = jnp.concatenate([edge_index[0], loop])
    dst = jnp.concatenate([edge_index[1], loop])
    ea = jnp.concatenate(
        [edge_attr,
         jnp.broadcast_to(edge_attr.mean(axis=0), (N, edge_attr.shape[1]))],
        axis=0)
    ids = jnp.arange(N, dtype=jnp.int32)
    Gs = (src[:, None] == ids[None, :]).astype(jnp.float32)
    Gd = (dst[:, None] == ids[None, :]).astype(jnp.float32)
    eye8 = jnp.eye(8, dtype=jnp.float32)
    A1 = (att1[:, :, None] * eye8[:, None, :]).reshape(512, 8)
    R1 = jnp.repeat(eye8, 64, axis=1)
    Wlr2 = jnp.concatenate([Wl2, Wr2], axis=1)
    return _run(x, Gs, Gd, ea, Wl1, Wr1, We1, A1, R1, b1.reshape(1, 512),
                Wlr2, We2, att2, b2.reshape(1, 512), Wih1,
                (bih1 + bhh1).reshape(1, 512), Whh1, Wih2, Whh2,
                (bih2 + bhh2).reshape(1, 256), Wfc, bfc.reshape(1, 10))


# speedup vs baseline: 1.3781x; 1.3781x over previous
import jax
import jax.numpy as jnp
from jax.experimental import pallas as pl

N = 33
E = 545
T = 512
TB = 32
KB = 2816


def _lrelu(v):
    return jnp.maximum(v, 0.2 * v)


def _elu(v):
    return jnp.where(v > 0, v, jnp.exp(v) - 1.0)


def _gat_body(x_ref, gs_ref, gd_ref, gdt_ref, ea_ref, wl1_ref, wr1_ref,
              we1_ref, a1_ref, r1_ref, b1_ref, wlr2_ref, we2_ref, att2_ref,
              b2_ref, out_ref):
    f32 = jnp.float32
    gs = gs_ref[...]
    gd = gd_ref[...]
    gdt = gdt_ref[...]
    ea = ea_ref[...]
    a1 = a1_ref[...]
    r1 = r1_ref[...]
    att2c = att2_ref[...]
    wl1 = wl1_ref[...]
    wr1 = wr1_ref[...]
    wlr2 = wlr2_ref[...]
    b1 = b1_ref[...]
    b2 = b2_ref[...]

    z58 = jnp.zeros((512, 8), f32)
    a1p = jnp.concatenate(
        [jnp.concatenate([a1, z58], axis=1),
         jnp.concatenate([z58, a1], axis=1)], axis=0)
    z85 = jnp.zeros((8, 512), f32)
    r1p = jnp.concatenate(
        [jnp.concatenate([r1, z85], axis=1),
         jnp.concatenate([z85, r1], axis=1)], axis=0)
    z51 = jnp.zeros((512, 1), f32)
    att2p = jnp.concatenate(
        [jnp.concatenate([att2c, z51], axis=1),
         jnp.concatenate([z51, att2c], axis=1)], axis=0)
    b1p = jnp.concatenate([b1, b1], axis=1)
    b2p = jnp.concatenate([b2, b2], axis=1)

    e1 = (ea[:, 0:1] * we1_ref[0:1, :] + ea[:, 1:2] * we1_ref[1:2, :]
          + ea[:, 2:3] * we1_ref[2:3, :])
    e2 = (ea[:, 0:1] * we2_ref[0:1, :] + ea[:, 1:2] * we2_ref[1:2, :]
          + ea[:, 2:3] * we2_ref[2:3, :])
    e1p = jnp.concatenate([e1, e1], axis=1)
    e2p = jnp.concatenate([e2, e2], axis=1)

    def per_pair(i, carry):
        t = 2 * i
        xt = jnp.concatenate([x_ref[t], x_ref[t + 1]], axis=1)
        xsd = jnp.dot(gs, xt, preferred_element_type=f32)
        xdd = jnp.dot(gd, xt, preferred_element_type=f32)
        xlsp = jnp.concatenate(
            [xsd[:, 0:1] * wl1[0:1, :] + xsd[:, 1:2] * wl1[1:2, :],
             xsd[:, 2:3] * wl1[0:1, :] + xsd[:, 3:4] * wl1[1:2, :]],
            axis=1)
        xrdp = jnp.concatenate(
            [xdd[:, 0:1] * wr1[0:1, :] + xdd[:, 1:2] * wr1[1:2, :],
             xdd[:, 2:3] * wr1[0:1, :] + xdd[:, 3:4] * wr1[1:2, :]],
            axis=1)
        m1 = _lrelu(xlsp + xrdp + e1p)
        al1 = jnp.dot(m1, a1p, preferred_element_type=f32)
        amax1 = jnp.max(al1, axis=0, keepdims=True)
        p1 = jnp.exp(al1 - amax1)
        s1 = jnp.dot(gdt, p1, preferred_element_type=f32)
        den1 = jnp.dot(gd, s1, preferred_element_type=f32)
        aln1 = p1 / (den1 + 1e-16)
        ab1 = jnp.dot(aln1, r1p, preferred_element_type=f32)
        h1p = _elu(jnp.dot(gdt, xlsp * ab1,
                           preferred_element_type=f32) + b1p)

        h1r = jnp.concatenate([h1p[:, 0:512], h1p[:, 512:1024]], axis=0)
        lrr = jnp.dot(h1r, wlr2, preferred_element_type=f32)
        xl2p = jnp.concatenate([lrr[0:N, 0:512], lrr[N:2 * N, 0:512]], axis=1)
        xr2p = jnp.concatenate([lrr[0:N, 512:1024], lrr[N:2 * N, 512:1024]],
                               axis=1)
        xls2 = jnp.dot(gs, xl2p, preferred_element_type=f32)
        xrd2 = jnp.dot(gd, xr2p, preferred_element_type=f32)
        m2 = _lrelu(xls2 + xrd2 + e2p)
        al2 = jnp.dot(m2, att2p, preferred_element_type=f32)
        amax2 = jnp.max(al2, axis=0, keepdims=True)
        p2 = jnp.exp(al2 - amax2)
        s2 = jnp.dot(gdt, p2, preferred_element_type=f32)
        den2 = jnp.dot(gd, s2, preferred_element_type=f32)
        aln2 = p2 / (den2 + 1e-16)
        msg2 = jnp.concatenate([xls2[:, 0:512] * aln2[:, 0:1],
                                xls2[:, 512:1024] * aln2[:, 1:2]], axis=1)
        h2p = _elu(jnp.dot(gdt, msg2,
                           preferred_element_type=f32) + b2p)
        out_ref[t] = h2p[:, 0:512]
        out_ref[t + 1] = h2p[:, 512:1024]
        return carry

    jax.lax.fori_loop(0, TB // 2, per_pair, 0)


def _proj_body(xl_ref, w_ref, b_ref, out_ref):
    k = pl.program_id(0)
    part = jnp.dot(xl_ref[...], w_ref[...], preferred_element_type=jnp.float32)

    @pl.when(k == 0)
    def _():
        out_ref[...] = part + b_ref[...]

    @pl.when(k > 0)
    def _():
        out_ref[...] += part


def _lstm_body(xp_ref, whh1_ref, wih2_ref, whh2_ref, b2_ref, wfc_ref,
               bfc_ref, out_ref):
    whh1 = whh1_ref[...]
    wih2 = wih2_ref[...]
    whh2 = whh2_ref[...]
    b2 = b2_ref[...]

    def outer(t8, carry):
        h1, c1, h2, c2 = carry
        blk = xp_ref[t8]
        for j in range(8):
            xt = blk[j:j + 1, :]
            g1 = xt + jnp.dot(h1, whh1, preferred_element_type=jnp.float32)
            i1 = jax.nn.sigmoid(g1[:, 0:128])
            f1 = jax.nn.sigmoid(g1[:, 128:256])
            gg1 = jnp.tanh(g1[:, 256:384])
            o1 = jax.nn.sigmoid(g1[:, 384:512])
            c1 = f1 * c1 + i1 * gg1
            h1 = o1 * jnp.tanh(c1)
            g2 = (jnp.dot(h1, wih2, preferred_element_type=jnp.float32)
                  + jnp.dot(h2, whh2, preferred_element_type=jnp.float32) + b2)
            i2 = jax.nn.sigmoid(g2[:, 0:64])
            f2 = jax.nn.sigmoid(g2[:, 64:128])
            gg2 = jnp.tanh(g2[:, 128:192])
            o2 = jax.nn.sigmoid(g2[:, 192:256])
            c2 = f2 * c2 + i2 * gg2
            h2 = o2 * jnp.tanh(c2)
        return (h1, c1, h2, c2)

    z1 = jnp.zeros((1, 128), jnp.float32)
    z2 = jnp.zeros((1, 64), jnp.float32)
    _, _, h2, _ = jax.lax.fori_loop(0, T // 8, outer, (z1, z1, z2, z2))
    out_ref[...] = (jnp.dot(h2, wfc_ref[...],
                            preferred_element_type=jnp.float32) + bfc_ref[...])


def _run(x, Gs, Gd, ea, Wl1, Wr1, We1, A1, R1, b1, Wlr2, We2, att2, b2,
         Wih1, bsum1, Whh1, Wih2, Whh2, bsum2, Wfc, bfc, interpret=False):
    const = lambda *_: tuple(0 for _ in range(2))
    X = pl.pallas_call(
        _gat_body,
        grid=(T // TB,),
        in_specs=[
            pl.BlockSpec((TB, N, 2), lambda i: (i, 0, 0)),
            pl.BlockSpec((E, N), lambda i: (0, 0)),
            pl.BlockSpec((E, N), lambda i: (0, 0)),
            pl.BlockSpec((N, E), lambda i: (0, 0)),
            pl.BlockSpec((E, 3), lambda i: (0, 0)),
            pl.BlockSpec((2, 512), lambda i: (0, 0)),
            pl.BlockSpec((2, 512), lambda i: (0, 0)),
            pl.BlockSpec((3, 512), lambda i: (0, 0)),
            pl.BlockSpec((512, 8), lambda i: (0, 0)),
            pl.BlockSpec((8, 512), lambda i: (0, 0)),
            pl.BlockSpec((1, 512), lambda i: (0, 0)),
            pl.BlockSpec((512, 1024), lambda i: (0, 0)),
            pl.BlockSpec((3, 512), lambda i: (0, 0)),
            pl.BlockSpec((512, 1), lambda i: (0, 0)),
            pl.BlockSpec((1, 512), lambda i: (0, 0)),
        ],
        out_specs=pl.BlockSpec((TB, N, 512), lambda i: (i, 0, 0)),
        out_shape=jax.ShapeDtypeStruct((T, N, 512), jnp.float32),
        interpret=interpret,
    )(x, Gs, Gd, Gd.T, ea, Wl1, Wr1, We1, A1, R1, b1, Wlr2, We2,
      att2.reshape(512, 1), b2)

    Xl = jnp.transpose(X, (1, 0, 2)).reshape(T, N * 512)

    Xp = pl.pallas_call(
        _proj_body,
        grid=(N * 512 // KB,),
        in_specs=[
            pl.BlockSpec((T, KB), lambda k: (0, k)),
            pl.BlockSpec((KB, 512), lambda k: (k, 0)),
            pl.BlockSpec((1, 512), lambda k: (0, 0)),
        ],
        out_specs=pl.BlockSpec((T, 512), lambda k: (0, 0)),
        out_shape=jax.ShapeDtypeStruct((T, 512), jnp.float32),
        interpret=interpret,
    )(Xl, Wih1, bsum1)

    out = pl.pallas_call(
        _lstm_body,
        interpret=interpret,
        out_shape=jax.ShapeDtypeStruct((1, 10), jnp.float32),
    )(Xp.reshape(T // 8, 8, 512), Whh1, Wih2, Whh2, bsum2, Wfc, bfc)
    return out


def kernel(x, edge_index, edge_attr, Wl1, Wr1, We1, att1, b1, Wl2, Wr2, We2,
           att2, b2, Wih1, Whh1, bih1, bhh1, Wih2, Whh2, bih2, bhh2, Wfc,
           bfc):
    loop = jnp.arange(N, dtype=edge_index.dtype)
    src = jnp.concatenate([edge_index[0], loop])
    dst = jnp.concatenate([edge_index[1], loop])
    ea = jnp.concatenate(
        [edge_attr,
         jnp.broadcast_to(edge_attr.mean(axis=0), (N, edge_attr.shape[1]))],
        axis=0)
    ids = jnp.arange(N, dtype=jnp.int32)
    Gs = (src[:, None] == ids[None, :]).astype(jnp.float32)
    Gd = (dst[:, None] == ids[None, :]).astype(jnp.float32)
    eye8 = jnp.eye(8, dtype=jnp.float32)
    A1 = (att1[:, :, None] * eye8[:, None, :]).reshape(512, 8)
    R1 = jnp.repeat(eye8, 64, axis=1)
    Wlr2 = jnp.concatenate([Wl2, Wr2], axis=1)
    return _run(x, Gs, Gd, ea, Wl1, Wr1, We1, A1, R1, b1.reshape(1, 512),
                Wlr2, We2, att2, b2.reshape(1, 512), Wih1,
                (bih1 + bhh1).reshape(1, 512), Whh1, Wih2, Whh2,
                (bih2 + bhh2).reshape(1, 256), Wfc, bfc.reshape(1, 10))


# speedup vs baseline: 1.6058x; 1.1652x over previous
import jax
import jax.numpy as jnp
from jax.experimental import pallas as pl

N = 33
E = 545
T = 512
TB = 32
KB = 2816


def _lrelu(v):
    return jnp.maximum(v, 0.2 * v)


def _elu(v):
    return jnp.where(v > 0, v, jnp.exp(v) - 1.0)


def _gat_body(x_ref, gs_ref, gd_ref, gdt_ref, ea_ref, wl1_ref, wr1_ref,
              we1_ref, a1_ref, r1_ref, b1_ref, wlr2_ref, we2_ref, att2_ref,
              b2_ref, out_ref):
    f32 = jnp.float32
    P = 4
    gs = gs_ref[...]
    gd = gd_ref[...]
    gdt = gdt_ref[...]
    ea = ea_ref[...]
    a1 = a1_ref[...]
    r1 = r1_ref[...]
    att2c = att2_ref[...]
    wl1 = wl1_ref[...]
    wr1 = wr1_ref[...]
    wlr2 = wlr2_ref[...]
    b1 = b1_ref[...]
    b2 = b2_ref[...]

    def blockdiag(m):
        rows, cols = m.shape
        zc = jnp.zeros((rows, cols), f32)
        outr = []
        for i in range(P):
            outr.append(jnp.concatenate(
                [m if j == i else zc for j in range(P)], axis=1))
        return jnp.concatenate(outr, axis=0)

    a1p = blockdiag(a1)
    r1p = blockdiag(r1)
    att2p = blockdiag(att2c)
    b1p = jnp.concatenate([b1] * P, axis=1)
    b2p = jnp.concatenate([b2] * P, axis=1)

    e1 = (ea[:, 0:1] * we1_ref[0:1, :] + ea[:, 1:2] * we1_ref[1:2, :]
          + ea[:, 2:3] * we1_ref[2:3, :])
    e2 = (ea[:, 0:1] * we2_ref[0:1, :] + ea[:, 1:2] * we2_ref[1:2, :]
          + ea[:, 2:3] * we2_ref[2:3, :])
    e1p = jnp.concatenate([e1] * P, axis=1)
    e2p = jnp.concatenate([e2] * P, axis=1)

    def per_pack(i, carry):
        t = P * i
        xt = jnp.concatenate([x_ref[t + j] for j in range(P)], axis=1)
        xsd = jnp.dot(gs, xt, preferred_element_type=f32)
        xdd = jnp.dot(gd, xt, preferred_element_type=f32)
        xlsp = jnp.concatenate(
            [xsd[:, 2 * j:2 * j + 1] * wl1[0:1, :]
             + xsd[:, 2 * j + 1:2 * j + 2] * wl1[1:2, :] for j in range(P)],
            axis=1)
        xrdp = jnp.concatenate(
            [xdd[:, 2 * j:2 * j + 1] * wr1[0:1, :]
             + xdd[:, 2 * j + 1:2 * j + 2] * wr1[1:2, :] for j in range(P)],
            axis=1)
        m1 = _lrelu(xlsp + xrdp + e1p)
        al1 = jnp.dot(m1, a1p, preferred_element_type=f32)
        amax1 = jnp.max(al1, axis=0, keepdims=True)
        p1 = jnp.exp(al1 - amax1)
        s1 = jnp.dot(gdt, p1, preferred_element_type=f32)
        den1 = jnp.dot(gd, s1, preferred_element_type=f32)
        aln1 = p1 / (den1 + 1e-16)
        ab1 = jnp.dot(aln1, r1p, preferred_element_type=f32)
        h1p = _elu(jnp.dot(gdt, xlsp * ab1,
                           preferred_element_type=f32) + b1p)

        h1r = jnp.concatenate(
            [h1p[:, 512 * j:512 * (j + 1)] for j in range(P)], axis=0)
        lrr = jnp.dot(h1r, wlr2, preferred_element_type=f32)
        xl2p = jnp.concatenate(
            [lrr[N * j:N * (j + 1), 0:512] for j in range(P)], axis=1)
        xr2p = jnp.concatenate(
            [lrr[N * j:N * (j + 1), 512:1024] for j in range(P)], axis=1)
        xls2 = jnp.dot(gs, xl2p, preferred_element_type=f32)
        xrd2 = jnp.dot(gd, xr2p, preferred_element_type=f32)
        m2 = _lrelu(xls2 + xrd2 + e2p)
        al2 = jnp.dot(m2, att2p, preferred_element_type=f32)
        amax2 = jnp.max(al2, axis=0, keepdims=True)
        p2 = jnp.exp(al2 - amax2)
        s2 = jnp.dot(gdt, p2, preferred_element_type=f32)
        den2 = jnp.dot(gd, s2, preferred_element_type=f32)
        aln2 = p2 / (den2 + 1e-16)
        msg2 = jnp.concatenate(
            [xls2[:, 512 * j:512 * (j + 1)] * aln2[:, j:j + 1]
             for j in range(P)], axis=1)
        h2p = _elu(jnp.dot(gdt, msg2,
                           preferred_element_type=f32) + b2p)
        for j in range(P):
            out_ref[t + j] = h2p[:, 512 * j:512 * (j + 1)]
        return carry

    jax.lax.fori_loop(0, TB // P, per_pack, 0)


def _proj_body(xl_ref, w_ref, b_ref, out_ref):
    k = pl.program_id(0)
    part = jnp.dot(xl_ref[...], w_ref[...], preferred_element_type=jnp.float32)

    @pl.when(k == 0)
    def _():
        out_ref[...] = part + b_ref[...]

    @pl.when(k > 0)
    def _():
        out_ref[...] += part


def _lstm_body(xp_ref, whh1_ref, wih2_ref, whh2_ref, b2_ref, wfc_ref,
               bfc_ref, out_ref):
    whh1 = whh1_ref[...]
    wih2 = wih2_ref[...]
    whh2 = whh2_ref[...]
    b2 = b2_ref[...]

    def outer(t8, carry):
        h1, c1, h2, c2 = carry
        blk = xp_ref[t8]
        for j in range(8):
            xt = blk[j:j + 1, :]
            g1 = xt + jnp.dot(h1, whh1, preferred_element_type=jnp.float32)
            i1 = jax.nn.sigmoid(g1[:, 0:128])
            f1 = jax.nn.sigmoid(g1[:, 128:256])
            gg1 = jnp.tanh(g1[:, 256:384])
            o1 = jax.nn.sigmoid(g1[:, 384:512])
            c1 = f1 * c1 + i1 * gg1
            h1 = o1 * jnp.tanh(c1)
            g2 = (jnp.dot(h1, wih2, preferred_element_type=jnp.float32)
                  + jnp.dot(h2, whh2, preferred_element_type=jnp.float32) + b2)
            i2 = jax.nn.sigmoid(g2[:, 0:64])
            f2 = jax.nn.sigmoid(g2[:, 64:128])
            gg2 = jnp.tanh(g2[:, 128:192])
            o2 = jax.nn.sigmoid(g2[:, 192:256])
            c2 = f2 * c2 + i2 * gg2
            h2 = o2 * jnp.tanh(c2)
        return (h1, c1, h2, c2)

    z1 = jnp.zeros((1, 128), jnp.float32)
    z2 = jnp.zeros((1, 64), jnp.float32)
    _, _, h2, _ = jax.lax.fori_loop(0, T // 8, outer, (z1, z1, z2, z2))
    out_ref[...] = (jnp.dot(h2, wfc_ref[...],
                            preferred_element_type=jnp.float32) + bfc_ref[...])


def _run(x, Gs, Gd, ea, Wl1, Wr1, We1, A1, R1, b1, Wlr2, We2, att2, b2,
         Wih1, bsum1, Whh1, Wih2, Whh2, bsum2, Wfc, bfc, interpret=False):
    const = lambda *_: tuple(0 for _ in range(2))
    X = pl.pallas_call(
        _gat_body,
        grid=(T // TB,),
        in_specs=[
            pl.BlockSpec((TB, N, 2), lambda i: (i, 0, 0)),
            pl.BlockSpec((E, N), lambda i: (0, 0)),
            pl.BlockSpec((E, N), lambda i: (0, 0)),
            pl.BlockSpec((N, E), lambda i: (0, 0)),
            pl.BlockSpec((E, 3), lambda i: (0, 0)),
            pl.BlockSpec((2, 512), lambda i: (0, 0)),
            pl.BlockSpec((2, 512), lambda i: (0, 0)),
            pl.BlockSpec((3, 512), lambda i: (0, 0)),
            pl.BlockSpec((512, 8), lambda i: (0, 0)),
            pl.BlockSpec((8, 512), lambda i: (0, 0)),
            pl.BlockSpec((1, 512), lambda i: (0, 0)),
            pl.BlockSpec((512, 1024), lambda i: (0, 0)),
            pl.BlockSpec((3, 512), lambda i: (0, 0)),
            pl.BlockSpec((512, 1), lambda i: (0, 0)),
            pl.BlockSpec((1, 512), lambda i: (0, 0)),
        ],
        out_specs=pl.BlockSpec((TB, N, 512), lambda i: (i, 0, 0)),
        out_shape=jax.ShapeDtypeStruct((T, N, 512), jnp.float32),
        interpret=interpret,
    )(x, Gs, Gd, Gd.T, ea, Wl1, Wr1, We1, A1, R1, b1, Wlr2, We2,
      att2.reshape(512, 1), b2)

    Xl = jnp.transpose(X, (1, 0, 2)).reshape(T, N * 512)

    Xp = pl.pallas_call(
        _proj_body,
        grid=(N * 512 // KB,),
        in_specs=[
            pl.BlockSpec((T, KB), lambda k: (0, k)),
            pl.BlockSpec((KB, 512), lambda k: (k, 0)),
            pl.BlockSpec((1, 512), lambda k: (0, 0)),
        ],
        out_specs=pl.BlockSpec((T, 512), lambda k: (0, 0)),
        out_shape=jax.ShapeDtypeStruct((T, 512), jnp.float32),
        interpret=interpret,
    )(Xl, Wih1, bsum1)

    out = pl.pallas_call(
        _lstm_body,
        interpret=interpret,
        out_shape=jax.ShapeDtypeStruct((1, 10), jnp.float32),
    )(Xp.reshape(T // 8, 8, 512), Whh1, Wih2, Whh2, bsum2, Wfc, bfc)
    return out


def kernel(x, edge_index, edge_attr, Wl1, Wr1, We1, att1, b1, Wl2, Wr2, We2,
           att2, b2, Wih1, Whh1, bih1, bhh1, Wih2, Whh2, bih2, bhh2, Wfc,
           bfc):
    loop = jnp.arange(N, dtype=edge_index.dtype)
    src = jnp.concatenate([edge_index[0], loop])
    dst = jnp.concatenate([edge_index[1], loop])
    ea = jnp.concatenate(
        [edge_attr,
         jnp.broadcast_to(edge_attr.mean(axis=0), (N, edge_attr.shape[1]))],
        axis=0)
    ids = jnp.arange(N, dtype=jnp.int32)
    Gs = (src[:, None] == ids[None, :]).astype(jnp.float32)
    Gd = (dst[:, None] == ids[None, :]).astype(jnp.float32)
    eye8 = jnp.eye(8, dtype=jnp.float32)
    A1 = (att1[:, :, None] * eye8[:, None, :]).reshape(512, 8)
    R1 = jnp.repeat(eye8, 64, axis=1)
    Wlr2 = jnp.concatenate([Wl2, Wr2], axis=1)
    return _run(x, Gs, Gd, ea, Wl1, Wr1, We1, A1, R1, b1.reshape(1, 512),
                Wlr2, We2, att2, b2.reshape(1, 512), Wih1,
                (bih1 + bhh1).reshape(1, 512), Whh1, Wih2, Whh2,
                (bih2 + bhh2).reshape(1, 256), Wfc, bfc.reshape(1, 10))


# speedup vs baseline: 1.6113x; 1.0035x over previous
import jax
import jax.numpy as jnp
from jax.experimental import pallas as pl

N = 33
E = 545
T = 512
TB = 32
KB = 2816


def _lrelu(v):
    return jnp.maximum(v, 0.2 * v)


def _elu(v):
    return jnp.where(v > 0, v, jnp.exp(v) - 1.0)


def _gat_body(x_ref, gs_ref, gd_ref, gdt_ref, ea_ref, wl1_ref, wr1_ref,
              we1_ref, a1_ref, r1_ref, b1_ref, wlr2_ref, we2_ref, att2_ref,
              b2_ref, out_ref):
    f32 = jnp.float32
    P = 4
    gs = gs_ref[...]
    gd = gd_ref[...]
    gdt = gdt_ref[...]
    ea = ea_ref[...]
    a1 = a1_ref[...]
    r1 = r1_ref[...]
    att2c = att2_ref[...]
    wl1 = wl1_ref[...]
    wr1 = wr1_ref[...]
    wlr2 = wlr2_ref[...]
    b1 = b1_ref[...]
    b2 = b2_ref[...]

    def blockdiag(m):
        rows, cols = m.shape
        zc = jnp.zeros((rows, cols), f32)
        outr = []
        for i in range(P):
            outr.append(jnp.concatenate(
                [m if j == i else zc for j in range(P)], axis=1))
        return jnp.concatenate(outr, axis=0)

    a1p = blockdiag(a1)
    r1p = blockdiag(r1)
    att2p = blockdiag(att2c)
    b1p = jnp.concatenate([b1] * P, axis=1)
    b2p = jnp.concatenate([b2] * P, axis=1)

    e1 = (ea[:, 0:1] * we1_ref[0:1, :] + ea[:, 1:2] * we1_ref[1:2, :]
          + ea[:, 2:3] * we1_ref[2:3, :])
    e2 = (ea[:, 0:1] * we2_ref[0:1, :] + ea[:, 1:2] * we2_ref[1:2, :]
          + ea[:, 2:3] * we2_ref[2:3, :])
    e1p = jnp.concatenate([e1] * P, axis=1)
    e2p = jnp.concatenate([e2] * P, axis=1)

    def per_pack(i, carry):
        t = P * i
        xt = jnp.concatenate([x_ref[t + j] for j in range(P)], axis=1)
        xsd = jnp.dot(gs, xt, preferred_element_type=f32)
        xdd = jnp.dot(gd, xt, preferred_element_type=f32)
        xlsp = jnp.concatenate(
            [xsd[:, 2 * j:2 * j + 1] * wl1[0:1, :]
             + xsd[:, 2 * j + 1:2 * j + 2] * wl1[1:2, :] for j in range(P)],
            axis=1)
        xrdp = jnp.concatenate(
            [xdd[:, 2 * j:2 * j + 1] * wr1[0:1, :]
             + xdd[:, 2 * j + 1:2 * j + 2] * wr1[1:2, :] for j in range(P)],
            axis=1)
        m1 = _lrelu(xlsp + xrdp + e1p)
        al1 = jnp.dot(m1, a1p, preferred_element_type=f32)
        amax1 = jnp.max(al1, axis=0, keepdims=True)
        p1 = jnp.exp(al1 - amax1)
        s1 = jnp.dot(gdt, p1, preferred_element_type=f32)
        den1 = jnp.dot(gd, s1, preferred_element_type=f32)
        aln1 = p1 / (den1 + 1e-16)
        ab1 = jnp.dot(aln1, r1p, preferred_element_type=f32)
        h1p = _elu(jnp.dot(gdt, xlsp * ab1,
                           preferred_element_type=f32) + b1p)

        h1r = jnp.concatenate(
            [h1p[:, 512 * j:512 * (j + 1)] for j in range(P)], axis=0)
        lrr = jnp.dot(h1r, wlr2, preferred_element_type=f32)
        xl2p = jnp.concatenate(
            [lrr[N * j:N * (j + 1), 0:512] for j in range(P)], axis=1)
        xr2p = jnp.concatenate(
            [lrr[N * j:N * (j + 1), 512:1024] for j in range(P)], axis=1)
        xls2 = jnp.dot(gs, xl2p, preferred_element_type=f32)
        xrd2 = jnp.dot(gd, xr2p, preferred_element_type=f32)
        m2 = _lrelu(xls2 + xrd2 + e2p)
        al2 = jnp.dot(m2, att2p, preferred_element_type=f32)
        amax2 = jnp.max(al2, axis=0, keepdims=True)
        p2 = jnp.exp(al2 - amax2)
        s2 = jnp.dot(gdt, p2, preferred_element_type=f32)
        den2 = jnp.dot(gd, s2, preferred_element_type=f32)
        aln2 = p2 / (den2 + 1e-16)
        msg2 = jnp.concatenate(
            [xls2[:, 512 * j:512 * (j + 1)] * aln2[:, j:j + 1]
             for j in range(P)], axis=1)
        h2p = _elu(jnp.dot(gdt, msg2,
                           preferred_element_type=f32) + b2p)
        for j in range(P):
            out_ref[t + j] = h2p[:, 512 * j:512 * (j + 1)]
        return carry

    jax.lax.fori_loop(0, TB // P, per_pack, 0)


def _proj_body(xl_ref, w_ref, b_ref, out_ref):
    k = pl.program_id(0)
    part = jnp.dot(xl_ref[...], w_ref[...], preferred_element_type=jnp.float32)

    @pl.when(k == 0)
    def _():
        out_ref[...] = part + b_ref[...]

    @pl.when(k > 0)
    def _():
        out_ref[...] += part


def _lstm_body(xp_ref, whh1_ref, wih2_ref, whh2_ref, b2_ref, wfc_ref,
               bfc_ref, out_ref):
    whh1 = whh1_ref[...]
    wih2 = wih2_ref[...]
    whh2 = whh2_ref[...]
    b2 = b2_ref[...]

    w12 = jnp.concatenate([wih2, whh2], axis=0)

    def outer(t8, carry):
        h1, c1, h2, c2 = carry
        blk = xp_ref[t8]
        for j in range(8):
            xt = blk[j:j + 1, :]
            g1 = xt + jnp.dot(h1, whh1, preferred_element_type=jnp.float32)
            i1 = jax.nn.sigmoid(g1[:, 0:128])
            f1 = jax.nn.sigmoid(g1[:, 128:256])
            gg1 = jnp.tanh(g1[:, 256:384])
            o1 = jax.nn.sigmoid(g1[:, 384:512])
            c1 = f1 * c1 + i1 * gg1
            h1 = o1 * jnp.tanh(c1)
            h12 = jnp.concatenate([h1, h2], axis=1)
            g2 = jnp.dot(h12, w12, preferred_element_type=jnp.float32) + b2
            i2 = jax.nn.sigmoid(g2[:, 0:64])
            f2 = jax.nn.sigmoid(g2[:, 64:128])
            gg2 = jnp.tanh(g2[:, 128:192])
            o2 = jax.nn.sigmoid(g2[:, 192:256])
            c2 = f2 * c2 + i2 * gg2
            h2 = o2 * jnp.tanh(c2)
        return (h1, c1, h2, c2)

    z1 = jnp.zeros((1, 128), jnp.float32)
    z2 = jnp.zeros((1, 64), jnp.float32)
    _, _, h2, _ = jax.lax.fori_loop(0, T // 8, outer, (z1, z1, z2, z2))
    out_ref[...] = (jnp.dot(h2, wfc_ref[...],
                            preferred_element_type=jnp.float32) + bfc_ref[...])


def _run(x, Gs, Gd, ea, Wl1, Wr1, We1, A1, R1, b1, Wlr2, We2, att2, b2,
         Wih1, bsum1, Whh1, Wih2, Whh2, bsum2, Wfc, bfc, interpret=False):
    const = lambda *_: tuple(0 for _ in range(2))
    X = pl.pallas_call(
        _gat_body,
        grid=(T // TB,),
        in_specs=[
            pl.BlockSpec((TB, N, 2), lambda i: (i, 0, 0)),
            pl.BlockSpec((E, N), lambda i: (0, 0)),
            pl.BlockSpec((E, N), lambda i: (0, 0)),
            pl.BlockSpec((N, E), lambda i: (0, 0)),
            pl.BlockSpec((E, 3), lambda i: (0, 0)),
            pl.BlockSpec((2, 512), lambda i: (0, 0)),
            pl.BlockSpec((2, 512), lambda i: (0, 0)),
            pl.BlockSpec((3, 512), lambda i: (0, 0)),
            pl.BlockSpec((512, 8), lambda i: (0, 0)),
            pl.BlockSpec((8, 512), lambda i: (0, 0)),
            pl.BlockSpec((1, 512), lambda i: (0, 0)),
            pl.BlockSpec((512, 1024), lambda i: (0, 0)),
            pl.BlockSpec((3, 512), lambda i: (0, 0)),
            pl.BlockSpec((512, 1), lambda i: (0, 0)),
            pl.BlockSpec((1, 512), lambda i: (0, 0)),
        ],
        out_specs=pl.BlockSpec((TB, N, 512), lambda i: (i, 0, 0)),
        out_shape=jax.ShapeDtypeStruct((T, N, 512), jnp.float32),
        interpret=interpret,
    )(x, Gs, Gd, Gd.T, ea, Wl1, Wr1, We1, A1, R1, b1, Wlr2, We2,
      att2.reshape(512, 1), b2)

    Xl = jnp.transpose(X, (1, 0, 2)).reshape(T, N * 512)

    Xp = pl.pallas_call(
        _proj_body,
        grid=(N * 512 // KB,),
        in_specs=[
            pl.BlockSpec((T, KB), lambda k: (0, k)),
            pl.BlockSpec((KB, 512), lambda k: (k, 0)),
            pl.BlockSpec((1, 512), lambda k: (0, 0)),
        ],
        out_specs=pl.BlockSpec((T, 512), lambda k: (0, 0)),
        out_shape=jax.ShapeDtypeStruct((T, 512), jnp.float32),
        interpret=interpret,
    )(Xl, Wih1, bsum1)

    out = pl.pallas_call(
        _lstm_body,
        interpret=interpret,
        out_shape=jax.ShapeDtypeStruct((1, 10), jnp.float32),
    )(Xp.reshape(T // 8, 8, 512), Whh1, Wih2, Whh2, bsum2, Wfc, bfc)
    return out


def kernel(x, edge_index, edge_attr, Wl1, Wr1, We1, att1, b1, Wl2, Wr2, We2,
           att2, b2, Wih1, Whh1, bih1, bhh1, Wih2, Whh2, bih2, bhh2, Wfc,
           bfc):
    loop = jnp.arange(N, dtype=edge_index.dtype)
    src = jnp.concatenate([edge_index[0], loop])
    dst = jnp.concatenate([edge_index[1], loop])
    ea = jnp.concatenate(
        [edge_attr,
         jnp.broadcast_to(edge_attr.mean(axis=0), (N, edge_attr.shape[1]))],
        axis=0)
    ids = jnp.arange(N, dtype=jnp.int32)
    Gs = (src[:, None] == ids[None, :]).astype(jnp.float32)
    Gd = (dst[:, None] == ids[None, :]).astype(jnp.float32)
    eye8 = jnp.eye(8, dtype=jnp.float32)
    A1 = (att1[:, :, None] * eye8[:, None, :]).reshape(512, 8)
    R1 = jnp.repeat(eye8, 64, axis=1)
    Wlr2 = jnp.concatenate([Wl2, Wr2], axis=1)
    return _run(x, Gs, Gd, ea, Wl1, Wr1, We1, A1, R1, b1.reshape(1, 512),
                Wlr2, We2, att2, b2.reshape(1, 512), Wih1,
                (bih1 + bhh1).reshape(1, 512), Whh1, Wih2, Whh2,
                (bih2 + bhh2).reshape(1, 256), Wfc, bfc.reshape(1, 10))
